# baseline (speedup 1.0000x reference)
import functools

import jax
import jax.numpy as jnp
from jax import lax
from jax.experimental import pallas as pl
from jax.experimental.pallas import tpu as pltpu
from jax.experimental.pallas import tpu_sc as plsc

T, R, N, E, D, H = 3, 2, 10000, 320000, 128, 128
HD = D // 2
HDI = HD // 2
NTILE = 16
NPAD = 10240
RPT = NPAD // NTILE
C = 128
EPAD = 321536
EPT = EPAD // NTILE
NCH = EPT // C

_BCAST_DNUMS = lax.GatherDimensionNumbers(
    offset_dims=(), collapsed_slice_dims=(0,), start_index_map=(0,))


def _bcast_lane(vec, lane):
    idx = jnp.full((16, 1), lane, jnp.int32)
    return lax.gather(vec, idx, _BCAST_DNUMS, (1,),
                      mode=lax.GatherScatterMode.PROMISE_IN_BOUNDS)


def _scale_chunk(in_ref, out_ref, w1d, k):
    for g in range(C // 16):
        wvec = w1d[pl.ds(k * C + g * 16, 16)]
        for u in range(16):
            wb = _bcast_lane(wvec, u)
            e = g * 16 + u
            for j in range(HDI // 16):
                v = in_ref[e, pl.ds(j * 16, 16)]
                ev = lax.bitcast_convert_type(v << 16, jnp.float32)
                od = lax.bitcast_convert_type(v & jnp.int32(-65536),
                                              jnp.float32)
                out_ref[e, pl.ds(j * 32, 16)] = ev * wb
                out_ref[e, pl.ds(j * 32 + 16, 16)] = od * wb


@functools.lru_cache(maxsize=None)
def _make_sc_pass(nx, with_deg):
    mesh = plsc.VectorSubcoreMesh(core_axis_name="c", subcore_axis_name="s")

    agg_t = jax.ShapeDtypeStruct((R, NPAD, HD), jnp.float32)
    out_type = agg_t
    scratch = [
        pltpu.VMEM_SHARED((NPAD, HD), jnp.float32),
        pltpu.VMEM((NCH, C), jnp.int32),
        pltpu.VMEM((NCH, C), jnp.int32),
        pltpu.VMEM((EPT,), jnp.float32),
        pltpu.VMEM((2, C, HDI), jnp.int32),
        pltpu.VMEM((2, C, HD), jnp.float32),
        pltpu.SemaphoreType.DMA((2,)),
        pltpu.SemaphoreType.DMA((2,)),
    ]
    if with_deg:
        out_type = [agg_t, jax.ShapeDtypeStruct((R, NPAD), jnp.float32)]
        scratch += [
            pltpu.VMEM_SHARED((NPAD,), jnp.float32),
            pltpu.VMEM((C,), jnp.float32),
        ]

    def sc_pass(*refs):
        if with_deg:
            (x_hbm, src_hbm, dst_hbm, w_hbm, z2_hbm, z1_hbm, ones_hbm,
             agg_out, deg_out, acc, sidx, didx, w1d, rin3, rout3, gsem, ssem,
             dacc, ones_v) = refs
        else:
            (x_hbm, src_hbm, dst_hbm, w_hbm, z2_hbm,
             agg_out, acc, sidx, didx, w1d, rin3, rout3, gsem, ssem) = refs
        c = lax.axis_index("c")
        s = lax.axis_index("s")

        pltpu.sync_copy(src_hbm.at[c, s], sidx)
        pltpu.sync_copy(dst_hbm.at[c, s], didx)
        pltpu.sync_copy(w_hbm.at[c, s], w1d)
        row0 = s * RPT
        pltpu.sync_copy(z2_hbm.at[pl.ds(row0, RPT)], acc.at[pl.ds(row0, RPT)])
        if with_deg:
            pltpu.sync_copy(ones_hbm, ones_v)
            pltpu.sync_copy(z1_hbm.at[pl.ds(row0, RPT)],
                            dacc.at[pl.ds(row0, RPT)])
        plsc.subcore_barrier()

        def issue_gather(k, b):
            pltpu.async_copy(x_hbm.at[sidx.at[k]], rin3.at[b], gsem.at[b])

        def wait_gather(k, b):
            pltpu.make_async_copy(x_hbm.at[sidx.at[k]], rin3.at[b],
                                  gsem.at[b]).wait()

        def issue_scatter(k, b):
            pltpu.async_copy(rout3.at[b], acc.at[didx.at[k]], ssem.at[b],
                             add=True)
            if with_deg:
                pltpu.async_copy(ones_v, dacc.at[didx.at[k]], ssem.at[b],
                                 add=True)

        def wait_scatter(k, b):
            pltpu.make_async_copy(rout3.at[b], acc.at[didx.at[k]],
                                  ssem.at[b]).wait()
            if with_deg:
                pltpu.make_async_copy(ones_v, dacc.at[didx.at[k]],
                                      ssem.at[b]).wait()

        issue_gather(0, 0)

        def chunk_body(k, carry):
            b = lax.rem(k, 2)
            nb = lax.rem(k + 1, 2)

            @pl.when(k >= 2)
            def _():
                wait_scatter(k, b)

            @pl.when(k + 1 < NCH)
            def _():
                issue_gather(k + 1, nb)

            wait_gather(k, b)
            _scale_chunk(rin3.at[b], rout3.at[b], w1d, k)
            issue_scatter(k, b)
            return carry

        lax.fori_loop(0, NCH, chunk_body, 0)

        def drain_body(i, carry):
            wait_scatter(NCH - 1, lax.rem(NCH - 2 + i, 2))
            return carry

        lax.fori_loop(0, 2, drain_body, 0)
        plsc.subcore_barrier()

        pltpu.sync_copy(acc.at[pl.ds(row0, RPT)],
                        agg_out.at[c, pl.ds(row0, RPT)])
        if with_deg:
            pltpu.sync_copy(dacc.at[pl.ds(row0, RPT)],
                            deg_out.at[c, pl.ds(row0, RPT)])

    return pl.kernel(sc_pass, out_type=out_type, mesh=mesh,
                     scratch_types=scratch,
                     compiler_params=pltpu.CompilerParams(
                         use_tc_tiling_on_sc=False))


BLK = 1024


def _conv_body(alo_ref, ahi_ref, deg_ref, wlo_ref, whi_ref, b_ref, out_ref):
    dg = jnp.maximum(deg_ref[...], 1.0)
    alo = alo_ref[...]
    ahi = ahi_ref[...]
    out = b_ref[...]
    for r in range(R):
        inv = (1.0 / dg[r])[:, None]
        out = out + jnp.dot(alo[r] * inv, wlo_ref[r],
                            preferred_element_type=jnp.float32)
        out = out + jnp.dot(ahi[r] * inv, whi_ref[r],
                            preferred_element_type=jnp.float32)
    out_ref[...] = jnp.maximum(out, 0.0)


def _conv_tc(alo, ahi, deg, wlo, whi, b):
    return pl.pallas_call(
        _conv_body,
        grid=(NPAD // BLK,),
        in_specs=[
            pl.BlockSpec((R, BLK, HD), lambda i: (0, i, 0)),
            pl.BlockSpec((R, BLK, HD), lambda i: (0, i, 0)),
            pl.BlockSpec((R, BLK), lambda i: (0, i)),
            pl.BlockSpec((R, HD, H), lambda i: (0, 0, 0)),
            pl.BlockSpec((R, HD, H), lambda i: (0, 0, 0)),
            pl.BlockSpec((1, H), lambda i: (0, 0)),
        ],
        out_specs=pl.BlockSpec((BLK, H), lambda i: (i, 0)),
        out_shape=jax.ShapeDtypeStruct((NPAD, H), jnp.float32),
    )(alo, ahi, deg, wlo, whi, b)


def _gru_body(x0r, x1r, x2r, wihr, whhr, bihr, bhhr, cw1r, cb1r, cw2r, cb2r,
              out_ref):
    wih = wihr[...]
    whh = whhr[...]
    bih = bihr[...]
    bhh = bhhr[...]
    h = jnp.zeros((BLK, H), jnp.float32)
    for xr in (x0r, x1r, x2r):
        xt = xr[...]
        gi = jnp.dot(xt, wih, preferred_element_type=jnp.float32) + bih
        gh = jnp.dot(h, whh, preferred_element_type=jnp.float32) + bhh
        rg = jax.nn.sigmoid(gi[:, :H] + gh[:, :H])
        zg = jax.nn.sigmoid(gi[:, H:2 * H] + gh[:, H:2 * H])
        ng = jnp.tanh(gi[:, 2 * H:] + rg * gh[:, 2 * H:])
        h = (1.0 - zg) * ng + zg * h
    zf = jnp.maximum(
        jnp.dot(h, cw1r[...], preferred_element_type=jnp.float32) + cb1r[...],
        0.0)
    out_ref[...] = (jnp.dot(zf, cw2r[...], preferred_element_type=jnp.float32)
                    + cb2r[...])


def _gru_cls_tc(h0, h1, h2, wihT, whhT, bih, bhh, cw1, cb1, cw2p, cb2):
    full = lambda shape: pl.BlockSpec(shape, lambda i: tuple(0 for _ in shape))
    blk = pl.BlockSpec((BLK, H), lambda i: (i, 0))
    return pl.pallas_call(
        _gru_body,
        grid=(NPAD // BLK,),
        in_specs=[blk, blk, blk,
                  full((H, 3 * H)), full((H, 3 * H)),
                  full((1, 3 * H)), full((1, 3 * H)),
                  full((H, H)), full((1, H)), full((H, H)), full((1, H))],
        out_specs=pl.BlockSpec((BLK, H), lambda i: (i, 0)),
        out_shape=jax.ShapeDtypeStruct((NPAD, H), jnp.float32),
    )(h0, h1, h2, wihT, whhT, bih, bhh, cw1, cb1, cw2p, cb2)


def kernel(feat_seq, edge_src, edge_dst, edge_w, W1, b1, W2, b2,
           gru_Wih, gru_Whh, gru_bih, gru_bhh, cls_W1, cls_b1, cls_W2,
           cls_b2):
    z2 = jnp.zeros((NPAD, HD), jnp.float32)
    z1 = jnp.zeros((NPAD,), jnp.float32)
    ones = jnp.ones((C,), jnp.float32)
    b1s = (b1[0] + b1[1]).reshape(1, H)
    b2s = (b2[0] + b2[1]).reshape(1, H)
    perm = []
    for j in (0, 1):
        perm += [j * 32 + 2 * i for i in range(16)]
        perm += [j * 32 + 2 * i + 1 for i in range(16)]
    w1lo, w1hi = W1[:, :HD, :][:, perm, :], W1[:, HD:, :][:, perm, :]
    w2lo, w2hi = W2[:, :HD, :][:, perm, :], W2[:, HD:, :][:, perm, :]

    def pack_half(x):
        xb = x.astype(jnp.bfloat16).reshape(-1, HDI, 2)
        return lax.bitcast_convert_type(xb, jnp.int32)

    sc_n_deg = _make_sc_pass(N, True)
    sc_n = _make_sc_pass(N, False)
    sc_p = _make_sc_pass(NPAD, False)

    npad_e = EPAD - E
    pad_idx = jnp.arange(npad_e, dtype=jnp.int32)
    pad_src = jnp.broadcast_to((pad_idx % N)[None, None], (T, R, npad_e))
    pad_dst = jnp.broadcast_to((N + pad_idx % (NPAD - N))[None, None],
                               (T, R, npad_e))
    src_all = jnp.concatenate([edge_src, pad_src], axis=2)
    dst_all = jnp.concatenate([edge_dst, pad_dst], axis=2)
    w_all = jnp.concatenate(
        [edge_w, jnp.zeros((T, R, npad_e), jnp.float32)], axis=2)

    hs = []
    for t in range(T):
        src4 = src_all[t].reshape(R, NTILE, NCH, C)
        dst4 = dst_all[t].reshape(R, NTILE, NCH, C)
        w3 = w_all[t].reshape(R, NTILE, EPT)
        x = feat_seq[t]
        a1lo, deg = sc_n_deg(pack_half(x[:, :HD]), src4, dst4, w3, z2, z1,
                             ones)
        a1hi = sc_n(pack_half(x[:, HD:]), src4, dst4, w3, z2)
        hl1 = _conv_tc(a1lo, a1hi, deg, w1lo, w1hi, b1s)
        a2lo = sc_p(pack_half(hl1[:, :HD]), src4, dst4, w3, z2)
        a2hi = sc_p(pack_half(hl1[:, HD:]), src4, dst4, w3, z2)
        hs.append(_conv_tc(a2lo, a2hi, deg, w2lo, w2hi, b2s))

    wihT = gru_Wih.T
    whhT = gru_Whh.T
    bih = gru_bih.reshape(1, 3 * H)
    bhh = gru_bhh.reshape(1, 3 * H)
    cb1 = cls_b1.reshape(1, H)
    cw2p = jnp.pad(cls_W2, ((0, 0), (0, H - 1)))
    cb2 = jnp.broadcast_to(cls_b2.reshape(1, 1), (1, H))
    logits = _gru_cls_tc(hs[0], hs[1], hs[2], wihT, whhT, bih, bhh,
                         cls_W1, cb1, cw2p, cb2)
    return logits[:N, 0]


# speedup vs baseline: 1.5268x; 1.5268x over previous
import functools

import jax
import jax.numpy as jnp
from jax import lax
from jax.experimental import pallas as pl
from jax.experimental.pallas import tpu as pltpu
from jax.experimental.pallas import tpu_sc as plsc

T, R, N, E, D, H = 3, 2, 10000, 320000, 128, 128
HD = D // 2
HDI = HD // 2
NTILE = 16
NPAD = 10240
RPT = NPAD // NTILE
C = 96
EPAD = 321024
EPT = EPAD // NTILE
NCH = EPT // C

_BCAST_DNUMS = lax.GatherDimensionNumbers(
    offset_dims=(), collapsed_slice_dims=(0,), start_index_map=(0,))


def _bcast_lane(vec, lane):
    idx = jnp.full((16, 1), lane, jnp.int32)
    return lax.gather(vec, idx, _BCAST_DNUMS, (1,),
                      mode=lax.GatherScatterMode.PROMISE_IN_BOUNDS)


def _scale_chunk(in_ref, out_ref, w1d, k):
    for g in range(C // 16):
        wvec = w1d[pl.ds(k * C + g * 16, 16)]
        for u in range(16):
            wb = _bcast_lane(wvec, u)
            e = g * 16 + u
            for j in range(HDI // 16):
                v = in_ref[e, pl.ds(j * 16, 16)]
                ev = lax.bitcast_convert_type(v << 16, jnp.float32)
                od = lax.bitcast_convert_type(v & jnp.int32(-65536),
                                              jnp.float32)
                out_ref[e, pl.ds(j * 32, 16)] = ev * wb
                out_ref[e, pl.ds(j * 32 + 16, 16)] = od * wb


@functools.lru_cache(maxsize=None)
def _make_sc_pass(nx, with_deg):
    mesh = plsc.VectorSubcoreMesh(core_axis_name="c", subcore_axis_name="s")

    agg_t = jax.ShapeDtypeStruct((R, NPAD, HD), jnp.float32)
    out_type = agg_t
    scratch = [
        pltpu.VMEM_SHARED((NPAD, HD), jnp.float32),
        pltpu.VMEM((NCH, C), jnp.int32),
        pltpu.VMEM((NCH, C), jnp.int32),
        pltpu.VMEM((EPT,), jnp.float32),
        pltpu.VMEM((2, C, HDI), jnp.int32),
        pltpu.VMEM((3, C, HD), jnp.float32),
        pltpu.SemaphoreType.DMA((2,)),
        pltpu.SemaphoreType.DMA((3,)),
    ]
    if with_deg:
        out_type = [agg_t, jax.ShapeDtypeStruct((R, NPAD), jnp.float32)]
        scratch += [
            pltpu.VMEM_SHARED((NPAD,), jnp.float32),
            pltpu.VMEM((C,), jnp.float32),
        ]

    def sc_pass(*refs):
        if with_deg:
            (x_hbm, src_hbm, dst_hbm, w_hbm, z2_hbm, z1_hbm, ones_hbm,
             agg_out, deg_out, acc, sidx, didx, w1d, rin3, rout3, gsem, ssem,
             dacc, ones_v) = refs
        else:
            (x_hbm, src_hbm, dst_hbm, w_hbm, z2_hbm,
             agg_out, acc, sidx, didx, w1d, rin3, rout3, gsem, ssem) = refs
        c = lax.axis_index("c")
        s = lax.axis_index("s")

        pltpu.sync_copy(src_hbm.at[c, s], sidx)
        pltpu.sync_copy(dst_hbm.at[c, s], didx)
        pltpu.sync_copy(w_hbm.at[c, s], w1d)
        row0 = s * RPT
        pltpu.sync_copy(z2_hbm.at[pl.ds(row0, RPT)], acc.at[pl.ds(row0, RPT)])
        if with_deg:
            pltpu.sync_copy(ones_hbm, ones_v)
            pltpu.sync_copy(z1_hbm.at[pl.ds(row0, RPT)],
                            dacc.at[pl.ds(row0, RPT)])
        plsc.subcore_barrier()

        def issue_gather(k, b):
            pltpu.async_copy(x_hbm.at[sidx.at[k]], rin3.at[b], gsem.at[b])

        def wait_gather(k, b):
            pltpu.make_async_copy(x_hbm.at[sidx.at[k]], rin3.at[b],
                                  gsem.at[b]).wait()

        def issue_scatter(k, b):
            pltpu.async_copy(rout3.at[b], acc.at[didx.at[k]], ssem.at[b],
                             add=True)
            if with_deg:
                pltpu.async_copy(ones_v, dacc.at[didx.at[k]], ssem.at[b],
                                 add=True)

        def wait_scatter(k, b):
            pltpu.make_async_copy(rout3.at[b], acc.at[didx.at[k]],
                                  ssem.at[b]).wait()
            if with_deg:
                pltpu.make_async_copy(ones_v, dacc.at[didx.at[k]],
                                      ssem.at[b]).wait()

        issue_gather(0, 0)

        def chunk_body(k, carry):
            bg = lax.rem(k, 2)
            nbg = lax.rem(k + 1, 2)
            bs = lax.rem(k, 3)

            @pl.when(k >= 3)
            def _():
                wait_scatter(k, bs)

            @pl.when(k + 1 < NCH)
            def _():
                issue_gather(k + 1, nbg)

            wait_gather(k, bg)
            _scale_chunk(rin3.at[bg], rout3.at[bs], w1d, k)
            issue_scatter(k, bs)
            return carry

        lax.fori_loop(0, NCH, chunk_body, 0)

        def drain_body(i, carry):
            wait_scatter(NCH - 1, lax.rem(NCH - 3 + i, 3))
            return carry

        lax.fori_loop(0, 3, drain_body, 0)
        plsc.subcore_barrier()

        pltpu.sync_copy(acc.at[pl.ds(row0, RPT)],
                        agg_out.at[c, pl.ds(row0, RPT)])
        if with_deg:
            pltpu.sync_copy(dacc.at[pl.ds(row0, RPT)],
                            deg_out.at[c, pl.ds(row0, RPT)])

    return pl.kernel(sc_pass, out_type=out_type, mesh=mesh,
                     scratch_types=scratch,
                     compiler_params=pltpu.CompilerParams(
                         use_tc_tiling_on_sc=False))


BLK = 1024


def _conv_body(alo_ref, ahi_ref, deg_ref, wlo_ref, whi_ref, b_ref, out_ref):
    dg = jnp.maximum(deg_ref[...], 1.0)
    alo = alo_ref[...]
    ahi = ahi_ref[...]
    out = b_ref[...]
    for r in range(R):
        inv = (1.0 / dg[r])[:, None]
        out = out + jnp.dot(alo[r] * inv, wlo_ref[r],
                            preferred_element_type=jnp.float32)
        out = out + jnp.dot(ahi[r] * inv, whi_ref[r],
                            preferred_element_type=jnp.float32)
    out_ref[...] = jnp.maximum(out, 0.0)


def _conv_tc(alo, ahi, deg, wlo, whi, b):
    return pl.pallas_call(
        _conv_body,
        grid=(NPAD // BLK,),
        in_specs=[
            pl.BlockSpec((R, BLK, HD), lambda i: (0, i, 0)),
            pl.BlockSpec((R, BLK, HD), lambda i: (0, i, 0)),
            pl.BlockSpec((R, BLK), lambda i: (0, i)),
            pl.BlockSpec((R, HD, H), lambda i: (0, 0, 0)),
            pl.BlockSpec((R, HD, H), lambda i: (0, 0, 0)),
            pl.BlockSpec((1, H), lambda i: (0, 0)),
        ],
        out_specs=pl.BlockSpec((BLK, H), lambda i: (i, 0)),
        out_shape=jax.ShapeDtypeStruct((NPAD, H), jnp.float32),
    )(alo, ahi, deg, wlo, whi, b)


def _gru_body(x0r, x1r, x2r, wihr, whhr, bihr, bhhr, cw1r, cb1r, cw2r, cb2r,
              out_ref):
    wih = wihr[...]
    whh = whhr[...]
    bih = bihr[...]
    bhh = bhhr[...]
    h = jnp.zeros((BLK, H), jnp.float32)
    for xr in (x0r, x1r, x2r):
        xt = xr[...]
        gi = jnp.dot(xt, wih, preferred_element_type=jnp.float32) + bih
        gh = jnp.dot(h, whh, preferred_element_type=jnp.float32) + bhh
        rg = jax.nn.sigmoid(gi[:, :H] + gh[:, :H])
        zg = jax.nn.sigmoid(gi[:, H:2 * H] + gh[:, H:2 * H])
        ng = jnp.tanh(gi[:, 2 * H:] + rg * gh[:, 2 * H:])
        h = (1.0 - zg) * ng + zg * h
    zf = jnp.maximum(
        jnp.dot(h, cw1r[...], preferred_element_type=jnp.float32) + cb1r[...],
        0.0)
    out_ref[...] = (jnp.dot(zf, cw2r[...], preferred_element_type=jnp.float32)
                    + cb2r[...])


def _gru_cls_tc(h0, h1, h2, wihT, whhT, bih, bhh, cw1, cb1, cw2p, cb2):
    full = lambda shape: pl.BlockSpec(shape, lambda i: tuple(0 for _ in shape))
    blk = pl.BlockSpec((BLK, H), lambda i: (i, 0))
    return pl.pallas_call(
        _gru_body,
        grid=(NPAD // BLK,),
        in_specs=[blk, blk, blk,
                  full((H, 3 * H)), full((H, 3 * H)),
                  full((1, 3 * H)), full((1, 3 * H)),
                  full((H, H)), full((1, H)), full((H, H)), full((1, H))],
        out_specs=pl.BlockSpec((BLK, H), lambda i: (i, 0)),
        out_shape=jax.ShapeDtypeStruct((NPAD, H), jnp.float32),
    )(h0, h1, h2, wihT, whhT, bih, bhh, cw1, cb1, cw2p, cb2)


def kernel(feat_seq, edge_src, edge_dst, edge_w, W1, b1, W2, b2,
           gru_Wih, gru_Whh, gru_bih, gru_bhh, cls_W1, cls_b1, cls_W2,
           cls_b2):
    z2 = jnp.zeros((NPAD, HD), jnp.float32)
    z1 = jnp.zeros((NPAD,), jnp.float32)
    ones = jnp.ones((C,), jnp.float32)
    b1s = (b1[0] + b1[1]).reshape(1, H)
    b2s = (b2[0] + b2[1]).reshape(1, H)
    perm = []
    for j in (0, 1):
        perm += [j * 32 + 2 * i for i in range(16)]
        perm += [j * 32 + 2 * i + 1 for i in range(16)]
    w1lo, w1hi = W1[:, :HD, :][:, perm, :], W1[:, HD:, :][:, perm, :]
    w2lo, w2hi = W2[:, :HD, :][:, perm, :], W2[:, HD:, :][:, perm, :]

    def pack_half(x):
        xb = x.astype(jnp.bfloat16).reshape(-1, HDI, 2)
        return lax.bitcast_convert_type(xb, jnp.int32)

    sc_n_deg = _make_sc_pass(N, True)
    sc_n = _make_sc_pass(N, False)
    sc_p = _make_sc_pass(NPAD, False)

    npad_e = EPAD - E
    pad_idx = jnp.arange(npad_e, dtype=jnp.int32)
    pad_src = jnp.broadcast_to((pad_idx % N)[None, None], (T, R, npad_e))
    pad_dst = jnp.broadcast_to((N + pad_idx % (NPAD - N))[None, None],
                               (T, R, npad_e))
    src_all = jnp.concatenate([edge_src, pad_src], axis=2)
    dst_all = jnp.concatenate([edge_dst, pad_dst], axis=2)
    w_all = jnp.concatenate(
        [edge_w, jnp.zeros((T, R, npad_e), jnp.float32)], axis=2)

    hs = []
    for t in range(T):
        src4 = src_all[t].reshape(R, NTILE, NCH, C)
        dst4 = dst_all[t].reshape(R, NTILE, NCH, C)
        w3 = w_all[t].reshape(R, NTILE, EPT)
        x = feat_seq[t]
        a1lo, deg = sc_n_deg(pack_half(x[:, :HD]), src4, dst4, w3, z2, z1,
                             ones)
        a1hi = sc_n(pack_half(x[:, HD:]), src4, dst4, w3, z2)
        hl1 = _conv_tc(a1lo, a1hi, deg, w1lo, w1hi, b1s)
        a2lo = sc_p(pack_half(hl1[:, :HD]), src4, dst4, w3, z2)
        a2hi = sc_p(pack_half(hl1[:, HD:]), src4, dst4, w3, z2)
        hs.append(_conv_tc(a2lo, a2hi, deg, w2lo, w2hi, b2s))

    wihT = gru_Wih.T
    whhT = gru_Whh.T
    bih = gru_bih.reshape(1, 3 * H)
    bhh = gru_bhh.reshape(1, 3 * H)
    cb1 = cls_b1.reshape(1, H)
    cw2p = jnp.pad(cls_W2, ((0, 0), (0, H - 1)))
    cb2 = jnp.broadcast_to(cls_b2.reshape(1, 1), (1, H))
    logits = _gru_cls_tc(hs[0], hs[1], hs[2], wihT, whhT, bih, bhh,
                         cls_W1, cb1, cw2p, cb2)
    return logits[:N, 0]


# speedup vs baseline: 2.7981x; 1.8327x over previous
import functools

import jax
import jax.numpy as jnp
from jax import lax
from jax.experimental import pallas as pl
from jax.experimental.pallas import tpu as pltpu
from jax.experimental.pallas import tpu_sc as plsc

T, R, N, E, D, H = 3, 2, 10000, 320000, 128, 128
HD = D // 2
HDI = HD // 2
NTILE = 16
NPAD = 10240
RPT = NPAD // NTILE
C = 128
EPAD = 321536
EPT = EPAD // NTILE
NCH = EPT // C

_BCAST_DNUMS = lax.GatherDimensionNumbers(
    offset_dims=(), collapsed_slice_dims=(0,), start_index_map=(0,))


def _bcast_lane(vec, lane):
    idx = jnp.full((16, 1), lane, jnp.int32)
    return lax.gather(vec, idx, _BCAST_DNUMS, (1,),
                      mode=lax.GatherScatterMode.PROMISE_IN_BOUNDS)


def _scale_chunk(rows_ref, w1d, k):
    for g in range(C // 16):
        wvec = w1d[pl.ds(k * C + g * 16, 16)]
        for u in range(16):
            wb = _bcast_lane(wvec, u)
            e = g * 16 + u
            for j in range(HD // 16):
                sl = pl.ds(j * 16, 16)
                rows_ref[e, sl] = rows_ref[e, sl] * wb


@functools.lru_cache(maxsize=None)
def _make_sc_pass(nx, with_deg):
    mesh = plsc.VectorSubcoreMesh(core_axis_name="c", subcore_axis_name="s")

    agg_t = jax.ShapeDtypeStruct((R, NPAD, HD), jnp.float32)
    out_type = agg_t
    scratch = [
        pltpu.VMEM_SHARED((NPAD, HD), jnp.float32),
        pltpu.VMEM((NCH, C), jnp.int32),
        pltpu.VMEM((NCH, C), jnp.int32),
        pltpu.VMEM((EPT,), jnp.float32),
        pltpu.VMEM((3, C, HD), jnp.float32),
        pltpu.SemaphoreType.DMA((3,)),
        pltpu.SemaphoreType.DMA((3,)),
    ]
    if with_deg:
        out_type = [agg_t, jax.ShapeDtypeStruct((R, NPAD), jnp.float32)]
        scratch += [
            pltpu.VMEM_SHARED((NPAD,), jnp.float32),
            pltpu.VMEM((C,), jnp.float32),
        ]

    def sc_pass(*refs):
        if with_deg:
            (x_hbm, src_hbm, dst_hbm, w_hbm, z2_hbm, z1_hbm, ones_hbm,
             agg_out, deg_out, acc, sidx, didx, w1d, rows3, gsem, ssem,
             dacc, ones_v) = refs
        else:
            (x_hbm, src_hbm, dst_hbm, w_hbm, z2_hbm,
             agg_out, acc, sidx, didx, w1d, rows3, gsem, ssem) = refs
        c = lax.axis_index("c")
        s = lax.axis_index("s")

        pltpu.sync_copy(src_hbm.at[c, s], sidx)
        pltpu.sync_copy(dst_hbm.at[c, s], didx)
        pltpu.sync_copy(w_hbm.at[c, s], w1d)
        row0 = s * RPT
        pltpu.sync_copy(z2_hbm.at[pl.ds(row0, RPT)], acc.at[pl.ds(row0, RPT)])
        if with_deg:
            pltpu.sync_copy(ones_hbm, ones_v)
            pltpu.sync_copy(z1_hbm.at[pl.ds(row0, RPT)],
                            dacc.at[pl.ds(row0, RPT)])
        plsc.subcore_barrier()

        def issue_gather(k, b):
            pltpu.async_copy(x_hbm.at[sidx.at[k]], rows3.at[b], gsem.at[b])

        def wait_gather(k, b):
            pltpu.make_async_copy(x_hbm.at[sidx.at[k]], rows3.at[b],
                                  gsem.at[b]).wait()

        def issue_scatter(k, b):
            pltpu.async_copy(rows3.at[b], acc.at[didx.at[k]], ssem.at[b],
                             add=True)
            if with_deg:
                pltpu.async_copy(ones_v, dacc.at[didx.at[k]], ssem.at[b],
                                 add=True)

        def wait_scatter(k, b):
            pltpu.make_async_copy(rows3.at[b], acc.at[didx.at[k]],
                                  ssem.at[b]).wait()
            if with_deg:
                pltpu.make_async_copy(ones_v, dacc.at[didx.at[k]],
                                      ssem.at[b]).wait()

        issue_gather(0, 0)

        def chunk_body(k, carry):
            b = lax.rem(k, 3)
            nb = lax.rem(k + 1, 3)

            @pl.when(k >= 2)
            def _():
                wait_scatter(k, nb)

            @pl.when(k + 1 < NCH)
            def _():
                issue_gather(k + 1, nb)

            wait_gather(k, b)
            _scale_chunk(rows3.at[b], w1d, k)
            issue_scatter(k, b)
            return carry

        lax.fori_loop(0, NCH, chunk_body, 0)

        def drain_body(i, carry):
            wait_scatter(NCH - 1, lax.rem(NCH - 2 + i, 3))
            return carry

        lax.fori_loop(0, 2, drain_body, 0)
        plsc.subcore_barrier()

        pltpu.sync_copy(acc.at[pl.ds(row0, RPT)],
                        agg_out.at[c, pl.ds(row0, RPT)])
        if with_deg:
            pltpu.sync_copy(dacc.at[pl.ds(row0, RPT)],
                            deg_out.at[c, pl.ds(row0, RPT)])

    return pl.kernel(sc_pass, out_type=out_type, mesh=mesh,
                     scratch_types=scratch,
                     compiler_params=pltpu.CompilerParams(
                         use_tc_tiling_on_sc=False))


BLK = 1024


def _conv_body(alo_ref, ahi_ref, deg_ref, wlo_ref, whi_ref, b_ref, out_ref):
    dg = jnp.maximum(deg_ref[...], 1.0)
    alo = alo_ref[...]
    ahi = ahi_ref[...]
    out = b_ref[...]
    for r in range(R):
        inv = (1.0 / dg[r])[:, None]
        out = out + jnp.dot(alo[r] * inv, wlo_ref[r],
                            preferred_element_type=jnp.float32)
        out = out + jnp.dot(ahi[r] * inv, whi_ref[r],
                            preferred_element_type=jnp.float32)
    out_ref[...] = jnp.maximum(out, 0.0)


def _conv_tc(alo, ahi, deg, wlo, whi, b):
    return pl.pallas_call(
        _conv_body,
        grid=(NPAD // BLK,),
        in_specs=[
            pl.BlockSpec((R, BLK, HD), lambda i: (0, i, 0)),
            pl.BlockSpec((R, BLK, HD), lambda i: (0, i, 0)),
            pl.BlockSpec((R, BLK), lambda i: (0, i)),
            pl.BlockSpec((R, HD, H), lambda i: (0, 0, 0)),
            pl.BlockSpec((R, HD, H), lambda i: (0, 0, 0)),
            pl.BlockSpec((1, H), lambda i: (0, 0)),
        ],
        out_specs=pl.BlockSpec((BLK, H), lambda i: (i, 0)),
        out_shape=jax.ShapeDtypeStruct((NPAD, H), jnp.float32),
    )(alo, ahi, deg, wlo, whi, b)


def _gru_body(x0r, x1r, x2r, wihr, whhr, bihr, bhhr, cw1r, cb1r, cw2r, cb2r,
              out_ref):
    wih = wihr[...]
    whh = whhr[...]
    bih = bihr[...]
    bhh = bhhr[...]
    h = jnp.zeros((BLK, H), jnp.float32)
    for xr in (x0r, x1r, x2r):
        xt = xr[...]
        gi = jnp.dot(xt, wih, preferred_element_type=jnp.float32) + bih
        gh = jnp.dot(h, whh, preferred_element_type=jnp.float32) + bhh
        rg = jax.nn.sigmoid(gi[:, :H] + gh[:, :H])
        zg = jax.nn.sigmoid(gi[:, H:2 * H] + gh[:, H:2 * H])
        ng = jnp.tanh(gi[:, 2 * H:] + rg * gh[:, 2 * H:])
        h = (1.0 - zg) * ng + zg * h
    zf = jnp.maximum(
        jnp.dot(h, cw1r[...], preferred_element_type=jnp.float32) + cb1r[...],
        0.0)
    out_ref[...] = (jnp.dot(zf, cw2r[...], preferred_element_type=jnp.float32)
                    + cb2r[...])


def _gru_cls_tc(h0, h1, h2, wihT, whhT, bih, bhh, cw1, cb1, cw2p, cb2):
    full = lambda shape: pl.BlockSpec(shape, lambda i: tuple(0 for _ in shape))
    blk = pl.BlockSpec((BLK, H), lambda i: (i, 0))
    return pl.pallas_call(
        _gru_body,
        grid=(NPAD // BLK,),
        in_specs=[blk, blk, blk,
                  full((H, 3 * H)), full((H, 3 * H)),
                  full((1, 3 * H)), full((1, 3 * H)),
                  full((H, H)), full((1, H)), full((H, H)), full((1, H))],
        out_specs=pl.BlockSpec((BLK, H), lambda i: (i, 0)),
        out_shape=jax.ShapeDtypeStruct((NPAD, H), jnp.float32),
    )(h0, h1, h2, wihT, whhT, bih, bhh, cw1, cb1, cw2p, cb2)


def kernel(feat_seq, edge_src, edge_dst, edge_w, W1, b1, W2, b2,
           gru_Wih, gru_Whh, gru_bih, gru_bhh, cls_W1, cls_b1, cls_W2,
           cls_b2):
    z2 = jnp.zeros((NPAD, HD), jnp.float32)
    z1 = jnp.zeros((NPAD,), jnp.float32)
    ones = jnp.ones((C,), jnp.float32)
    b1s = (b1[0] + b1[1]).reshape(1, H)
    b2s = (b2[0] + b2[1]).reshape(1, H)
    w1lo, w1hi = W1[:, :HD, :], W1[:, HD:, :]
    w2lo, w2hi = W2[:, :HD, :], W2[:, HD:, :]

    sc_n_deg = _make_sc_pass(N, True)
    sc_n = _make_sc_pass(N, False)
    sc_p = _make_sc_pass(NPAD, False)

    npad_e = EPAD - E
    pad_idx = jnp.arange(npad_e, dtype=jnp.int32)
    pad_src = jnp.broadcast_to((pad_idx % N)[None, None], (T, R, npad_e))
    pad_dst = jnp.broadcast_to((N + pad_idx % (NPAD - N))[None, None],
                               (T, R, npad_e))
    src_all = jnp.concatenate([edge_src, pad_src], axis=2)
    dst_all = jnp.concatenate([edge_dst, pad_dst], axis=2)
    w_all = jnp.concatenate(
        [edge_w, jnp.zeros((T, R, npad_e), jnp.float32)], axis=2)

    hs = []
    for t in range(T):
        src4 = src_all[t].reshape(R, NTILE, NCH, C)
        dst4 = dst_all[t].reshape(R, NTILE, NCH, C)
        w3 = w_all[t].reshape(R, NTILE, EPT)
        x = feat_seq[t]
        a1lo, deg = sc_n_deg(x[:, :HD], src4, dst4, w3, z2, z1, ones)
        a1hi = sc_n(x[:, HD:], src4, dst4, w3, z2)
        hl1 = _conv_tc(a1lo, a1hi, deg, w1lo, w1hi, b1s)
        a2lo = sc_p(hl1[:, :HD], src4, dst4, w3, z2)
        a2hi = sc_p(hl1[:, HD:], src4, dst4, w3, z2)
        hs.append(_conv_tc(a2lo, a2hi, deg, w2lo, w2hi, b2s))

    wihT = gru_Wih.T
    whhT = gru_Whh.T
    bih = gru_bih.reshape(1, 3 * H)
    bhh = gru_bhh.reshape(1, 3 * H)
    cb1 = cls_b1.reshape(1, H)
    cw2p = jnp.pad(cls_W2, ((0, 0), (0, H - 1)))
    cb2 = jnp.broadcast_to(cls_b2.reshape(1, 1), (1, H))
    logits = _gru_cls_tc(hs[0], hs[1], hs[2], wihT, whhT, bih, bhh,
                         cls_W1, cb1, cw2p, cb2)
    return logits[:N, 0]


# speedup vs baseline: 2.9707x; 1.0617x over previous
import functools

import jax
import jax.numpy as jnp
from jax import lax
from jax.experimental import pallas as pl
from jax.experimental.pallas import tpu as pltpu
from jax.experimental.pallas import tpu_sc as plsc

T, R, N, E, D, H = 3, 2, 10000, 320000, 128, 128
HD = D // 2
HDI = HD // 2
NTILE = 16
NPAD = 10240
RPT = NPAD // NTILE
C = 128
EPAD = 321536
EPT = EPAD // NTILE
NCH = EPT // C

_BCAST_DNUMS = lax.GatherDimensionNumbers(
    offset_dims=(), collapsed_slice_dims=(0,), start_index_map=(0,))


def _bcast_lane(vec, lane):
    idx = jnp.full((16, 1), lane, jnp.int32)
    return lax.gather(vec, idx, _BCAST_DNUMS, (1,),
                      mode=lax.GatherScatterMode.PROMISE_IN_BOUNDS)


def _scale_chunk(rows_ref, w1d, k):
    for g in range(C // 16):
        wvec = w1d[pl.ds(k * C + g * 16, 16)]
        for u in range(16):
            wb = _bcast_lane(wvec, u)
            e = g * 16 + u
            for j in range(HD // 16):
                sl = pl.ds(j * 16, 16)
                rows_ref[e, sl] = rows_ref[e, sl] * wb


@functools.lru_cache(maxsize=None)
def _make_sc_pass(nx, with_deg):
    mesh = plsc.VectorSubcoreMesh(core_axis_name="c", subcore_axis_name="s")

    agg_t = jax.ShapeDtypeStruct((R, NPAD, D), jnp.float32)
    out_type = agg_t
    scratch = [
        pltpu.VMEM_SHARED((NPAD, HD), jnp.float32),
        pltpu.VMEM((NCH, C), jnp.int32),
        pltpu.VMEM((NCH, C), jnp.int32),
        pltpu.VMEM((EPT,), jnp.float32),
        pltpu.VMEM((3, C, HD), jnp.float32),
        pltpu.SemaphoreType.DMA((3,)),
        pltpu.SemaphoreType.DMA((3,)),
    ]
    if with_deg:
        out_type = [agg_t, jax.ShapeDtypeStruct((R, NPAD), jnp.float32)]
        scratch += [
            pltpu.VMEM_SHARED((NPAD,), jnp.float32),
            pltpu.VMEM((C,), jnp.float32),
        ]

    def sc_pass(*refs):
        if with_deg:
            (xlo_hbm, xhi_hbm, src_hbm, dst_hbm, w_hbm, z2_hbm, z1_hbm,
             ones_hbm, agg_out, deg_out, acc, sidx, didx, w1d, rows3, gsem,
             ssem, dacc, ones_v) = refs
        else:
            (xlo_hbm, xhi_hbm, src_hbm, dst_hbm, w_hbm, z2_hbm,
             agg_out, acc, sidx, didx, w1d, rows3, gsem, ssem) = refs
        c = lax.axis_index("c")
        s = lax.axis_index("s")
        row0 = s * RPT

        pltpu.sync_copy(src_hbm.at[c, s], sidx)
        pltpu.sync_copy(dst_hbm.at[c, s], didx)
        pltpu.sync_copy(w_hbm.at[c, s], w1d)
        if with_deg:
            pltpu.sync_copy(ones_hbm, ones_v)
            pltpu.sync_copy(z1_hbm.at[pl.ds(row0, RPT)],
                            dacc.at[pl.ds(row0, RPT)])

        def run_half(x_hbm, half, do_deg):
            pltpu.sync_copy(z2_hbm.at[pl.ds(row0, RPT)],
                            acc.at[pl.ds(row0, RPT)])
            plsc.subcore_barrier()

            def issue_gather(k, b):
                pltpu.async_copy(x_hbm.at[sidx.at[k]], rows3.at[b],
                                 gsem.at[b])

            def wait_gather(k, b):
                pltpu.make_async_copy(x_hbm.at[sidx.at[k]], rows3.at[b],
                                      gsem.at[b]).wait()

            def issue_scatter(k, b):
                pltpu.async_copy(rows3.at[b], acc.at[didx.at[k]], ssem.at[b],
                                 add=True)
                if do_deg:
                    pltpu.async_copy(ones_v, dacc.at[didx.at[k]], ssem.at[b],
                                     add=True)

            def wait_scatter(k, b):
                pltpu.make_async_copy(rows3.at[b], acc.at[didx.at[k]],
                                      ssem.at[b]).wait()
                if do_deg:
                    pltpu.make_async_copy(ones_v, dacc.at[didx.at[k]],
                                          ssem.at[b]).wait()

            issue_gather(0, 0)

            def chunk_body(k, carry):
                b = lax.rem(k, 3)
                nb = lax.rem(k + 1, 3)

                @pl.when(k >= 2)
                def _():
                    wait_scatter(k, nb)

                @pl.when(k + 1 < NCH)
                def _():
                    issue_gather(k + 1, nb)

                wait_gather(k, b)
                _scale_chunk(rows3.at[b], w1d, k)
                issue_scatter(k, b)
                return carry

            lax.fori_loop(0, NCH, chunk_body, 0)

            def drain_body(i, carry):
                wait_scatter(NCH - 1, lax.rem(NCH - 2 + i, 3))
                return carry

            lax.fori_loop(0, 2, drain_body, 0)
            plsc.subcore_barrier()

            pltpu.sync_copy(
                acc.at[pl.ds(row0, RPT)],
                agg_out.at[c, pl.ds(row0, RPT), pl.ds(half * HD, HD)])
            if do_deg:
                pltpu.sync_copy(dacc.at[pl.ds(row0, RPT)],
                                deg_out.at[c, pl.ds(row0, RPT)])
            plsc.subcore_barrier()

        run_half(xlo_hbm, 0, with_deg)
        run_half(xhi_hbm, 1, False)

    return pl.kernel(sc_pass, out_type=out_type, mesh=mesh,
                     scratch_types=scratch,
                     compiler_params=pltpu.CompilerParams(
                         use_tc_tiling_on_sc=False))


BLK = 1024


def _conv_body(agg_ref, deg_ref, w_ref, b_ref, out_ref):
    dg = jnp.maximum(deg_ref[...], 1.0)
    a = agg_ref[...]
    out = b_ref[...]
    for r in range(R):
        inv = (1.0 / dg[r])[:, None]
        out = out + jnp.dot(a[r] * inv, w_ref[r],
                            preferred_element_type=jnp.float32)
    out_ref[...] = jnp.maximum(out, 0.0)


def _conv_tc(agg, deg, w, b):
    return pl.pallas_call(
        _conv_body,
        grid=(NPAD // BLK,),
        in_specs=[
            pl.BlockSpec((R, BLK, D), lambda i: (0, i, 0)),
            pl.BlockSpec((R, BLK), lambda i: (0, i)),
            pl.BlockSpec((R, D, H), lambda i: (0, 0, 0)),
            pl.BlockSpec((1, H), lambda i: (0, 0)),
        ],
        out_specs=pl.BlockSpec((BLK, H), lambda i: (i, 0)),
        out_shape=jax.ShapeDtypeStruct((NPAD, H), jnp.float32),
    )(agg, deg, w, b)


def _gru_body(x0r, x1r, x2r, wihr, whhr, bihr, bhhr, cw1r, cb1r, cw2r, cb2r,
              out_ref):
    wih = wihr[...]
    whh = whhr[...]
    bih = bihr[...]
    bhh = bhhr[...]
    h = jnp.zeros((BLK, H), jnp.float32)
    for xr in (x0r, x1r, x2r):
        xt = xr[...]
        gi = jnp.dot(xt, wih, preferred_element_type=jnp.float32) + bih
        gh = jnp.dot(h, whh, preferred_element_type=jnp.float32) + bhh
        rg = jax.nn.sigmoid(gi[:, :H] + gh[:, :H])
        zg = jax.nn.sigmoid(gi[:, H:2 * H] + gh[:, H:2 * H])
        ng = jnp.tanh(gi[:, 2 * H:] + rg * gh[:, 2 * H:])
        h = (1.0 - zg) * ng + zg * h
    zf = jnp.maximum(
        jnp.dot(h, cw1r[...], preferred_element_type=jnp.float32) + cb1r[...],
        0.0)
    out_ref[...] = (jnp.dot(zf, cw2r[...], preferred_element_type=jnp.float32)
                    + cb2r[...])


def _gru_cls_tc(h0, h1, h2, wihT, whhT, bih, bhh, cw1, cb1, cw2p, cb2):
    full = lambda shape: pl.BlockSpec(shape, lambda i: tuple(0 for _ in shape))
    blk = pl.BlockSpec((BLK, H), lambda i: (i, 0))
    return pl.pallas_call(
        _gru_body,
        grid=(NPAD // BLK,),
        in_specs=[blk, blk, blk,
                  full((H, 3 * H)), full((H, 3 * H)),
                  full((1, 3 * H)), full((1, 3 * H)),
                  full((H, H)), full((1, H)), full((H, H)), full((1, H))],
        out_specs=pl.BlockSpec((BLK, H), lambda i: (i, 0)),
        out_shape=jax.ShapeDtypeStruct((NPAD, H), jnp.float32),
    )(h0, h1, h2, wihT, whhT, bih, bhh, cw1, cb1, cw2p, cb2)


def kernel(feat_seq, edge_src, edge_dst, edge_w, W1, b1, W2, b2,
           gru_Wih, gru_Whh, gru_bih, gru_bhh, cls_W1, cls_b1, cls_W2,
           cls_b2):
    z2 = jnp.zeros((NPAD, HD), jnp.float32)
    z1 = jnp.zeros((NPAD,), jnp.float32)
    ones = jnp.ones((C,), jnp.float32)
    b1s = (b1[0] + b1[1]).reshape(1, H)
    b2s = (b2[0] + b2[1]).reshape(1, H)

    sc_n_deg = _make_sc_pass(N, True)
    sc_p = _make_sc_pass(NPAD, False)

    npad_e = EPAD - E
    pad_idx = jnp.arange(npad_e, dtype=jnp.int32)
    pad_src = jnp.broadcast_to((pad_idx % N)[None, None], (T, R, npad_e))
    pad_dst = jnp.broadcast_to((N + pad_idx % (NPAD - N))[None, None],
                               (T, R, npad_e))
    src_all = jnp.concatenate([edge_src, pad_src], axis=2)
    dst_all = jnp.concatenate([edge_dst, pad_dst], axis=2)
    w_all = jnp.concatenate(
        [edge_w, jnp.zeros((T, R, npad_e), jnp.float32)], axis=2)

    hs = []
    for t in range(T):
        src4 = src_all[t].reshape(R, NTILE, NCH, C)
        dst4 = dst_all[t].reshape(R, NTILE, NCH, C)
        w3 = w_all[t].reshape(R, NTILE, EPT)
        x = feat_seq[t]
        a1, deg = sc_n_deg(x[:, :HD], x[:, HD:], src4, dst4, w3, z2, z1,
                           ones)
        hl1 = _conv_tc(a1, deg, W1, b1s)
        a2 = sc_p(hl1[:, :HD], hl1[:, HD:], src4, dst4, w3, z2)
        hs.append(_conv_tc(a2, deg, W2, b2s))

    wihT = gru_Wih.T
    whhT = gru_Whh.T
    bih = gru_bih.reshape(1, 3 * H)
    bhh = gru_bhh.reshape(1, 3 * H)
    cb1 = cls_b1.reshape(1, H)
    cw2p = jnp.pad(cls_W2, ((0, 0), (0, H - 1)))
    cb2 = jnp.broadcast_to(cls_b2.reshape(1, 1), (1, H))
    logits = _gru_cls_tc(hs[0], hs[1], hs[2], wihT, whhT, bih, bhh,
                         cls_W1, cb1, cw2p, cb2)
    return logits[:N, 0]


# speedup vs baseline: 3.2376x; 1.0898x over previous
import functools

import jax
import jax.numpy as jnp
from jax import lax
from jax.experimental import pallas as pl
from jax.experimental.pallas import tpu as pltpu
from jax.experimental.pallas import tpu_sc as plsc

T, R, N, E, D, H = 3, 2, 10000, 320000, 128, 128
HD = D // 2
HDI = HD // 2
NTILE = 16
NPAD = 10240
RPT = NPAD // NTILE
C = 112
EPAD = 320768
EPT = EPAD // NTILE
NCH = EPT // C

_BCAST_DNUMS = lax.GatherDimensionNumbers(
    offset_dims=(), collapsed_slice_dims=(0,), start_index_map=(0,))


def _bcast_lane(vec, lane):
    idx = jnp.full((16, 1), lane, jnp.int32)
    return lax.gather(vec, idx, _BCAST_DNUMS, (1,),
                      mode=lax.GatherScatterMode.PROMISE_IN_BOUNDS)


def _scale_chunk(rows_ref, w1d, k):
    for g in range(C // 16):
        wvec = w1d[pl.ds(k * C + g * 16, 16)]
        for u in range(16):
            wb = _bcast_lane(wvec, u)
            e = g * 16 + u
            for j in range(HD // 16):
                sl = pl.ds(j * 16, 16)
                rows_ref[e, sl] = rows_ref[e, sl] * wb


@functools.lru_cache(maxsize=None)
def _make_sc_pass(nx, with_deg):
    mesh = plsc.VectorSubcoreMesh(core_axis_name="c", subcore_axis_name="s")

    agg_t = jax.ShapeDtypeStruct((R, NPAD, D), jnp.float32)
    out_type = agg_t
    scratch = [
        pltpu.VMEM_SHARED((NPAD, HD), jnp.float32),
        pltpu.VMEM((NCH, C), jnp.int32),
        pltpu.VMEM((NCH, C), jnp.int32),
        pltpu.VMEM((EPT,), jnp.float32),
        pltpu.VMEM((4, C, HD), jnp.float32),
        pltpu.SemaphoreType.DMA((4,)),
        pltpu.SemaphoreType.DMA((4,)),
    ]
    if with_deg:
        out_type = [agg_t, jax.ShapeDtypeStruct((R, NPAD), jnp.float32)]
        scratch += [
            pltpu.VMEM_SHARED((NPAD,), jnp.float32),
            pltpu.VMEM((C,), jnp.float32),
        ]

    def sc_pass(*refs):
        if with_deg:
            (xlo_hbm, xhi_hbm, src_hbm, dst_hbm, w_hbm, z2_hbm, z1_hbm,
             ones_hbm, agg_out, deg_out, acc, sidx, didx, w1d, rows3, gsem,
             ssem, dacc, ones_v) = refs
        else:
            (xlo_hbm, xhi_hbm, src_hbm, dst_hbm, w_hbm, z2_hbm,
             agg_out, acc, sidx, didx, w1d, rows3, gsem, ssem) = refs
        c = lax.axis_index("c")
        s = lax.axis_index("s")
        row0 = s * RPT

        pltpu.sync_copy(src_hbm.at[c, s], sidx)
        pltpu.sync_copy(dst_hbm.at[c, s], didx)
        pltpu.sync_copy(w_hbm.at[c, s], w1d)
        if with_deg:
            pltpu.sync_copy(ones_hbm, ones_v)
            pltpu.sync_copy(z1_hbm.at[pl.ds(row0, RPT)],
                            dacc.at[pl.ds(row0, RPT)])

        def run_half(x_hbm, half, do_deg):
            pltpu.sync_copy(z2_hbm.at[pl.ds(row0, RPT)],
                            acc.at[pl.ds(row0, RPT)])
            plsc.subcore_barrier()

            def issue_gather(k, b):
                pltpu.async_copy(x_hbm.at[sidx.at[k]], rows3.at[b],
                                 gsem.at[b])

            def wait_gather(k, b):
                pltpu.make_async_copy(x_hbm.at[sidx.at[k]], rows3.at[b],
                                      gsem.at[b]).wait()

            def issue_scatter(k, b):
                pltpu.async_copy(rows3.at[b], acc.at[didx.at[k]], ssem.at[b],
                                 add=True)
                if do_deg:
                    pltpu.async_copy(ones_v, dacc.at[didx.at[k]], ssem.at[b],
                                     add=True)

            def wait_scatter(k, b):
                pltpu.make_async_copy(rows3.at[b], acc.at[didx.at[k]],
                                      ssem.at[b]).wait()
                if do_deg:
                    pltpu.make_async_copy(ones_v, dacc.at[didx.at[k]],
                                          ssem.at[b]).wait()

            issue_gather(0, 0)
            issue_gather(1, 1)

            def chunk_body(k, carry):
                b = lax.rem(k, 4)
                b2 = lax.rem(k + 2, 4)

                @pl.when(k >= 2)
                def _():
                    wait_scatter(k, b2)

                @pl.when(k + 2 < NCH)
                def _():
                    issue_gather(k + 2, b2)

                wait_gather(k, b)
                _scale_chunk(rows3.at[b], w1d, k)
                issue_scatter(k, b)
                return carry

            lax.fori_loop(0, NCH, chunk_body, 0)

            def drain_body(i, carry):
                wait_scatter(NCH - 1, lax.rem(NCH - 2 + i, 4))
                return carry

            lax.fori_loop(0, 2, drain_body, 0)
            plsc.subcore_barrier()

            pltpu.sync_copy(
                acc.at[pl.ds(row0, RPT)],
                agg_out.at[c, pl.ds(row0, RPT), pl.ds(half * HD, HD)])
            if do_deg:
                pltpu.sync_copy(dacc.at[pl.ds(row0, RPT)],
                                deg_out.at[c, pl.ds(row0, RPT)])
            plsc.subcore_barrier()

        run_half(xlo_hbm, 0, with_deg)
        run_half(xhi_hbm, 1, False)

    return pl.kernel(sc_pass, out_type=out_type, mesh=mesh,
                     scratch_types=scratch,
                     compiler_params=pltpu.CompilerParams(
                         use_tc_tiling_on_sc=False))


BLK = 1024


def _conv_body(agg_ref, deg_ref, w_ref, b_ref, out_ref):
    dg = jnp.maximum(deg_ref[...], 1.0)
    a = agg_ref[...]
    out = b_ref[...]
    for r in range(R):
        inv = (1.0 / dg[r])[:, None]
        out = out + jnp.dot(a[r] * inv, w_ref[r],
                            preferred_element_type=jnp.float32)
    out_ref[...] = jnp.maximum(out, 0.0)


def _conv_tc(agg, deg, w, b):
    return pl.pallas_call(
        _conv_body,
        grid=(NPAD // BLK,),
        in_specs=[
            pl.BlockSpec((R, BLK, D), lambda i: (0, i, 0)),
            pl.BlockSpec((R, BLK), lambda i: (0, i)),
            pl.BlockSpec((R, D, H), lambda i: (0, 0, 0)),
            pl.BlockSpec((1, H), lambda i: (0, 0)),
        ],
        out_specs=pl.BlockSpec((BLK, H), lambda i: (i, 0)),
        out_shape=jax.ShapeDtypeStruct((NPAD, H), jnp.float32),
    )(agg, deg, w, b)


def _gru_body(x0r, x1r, x2r, wihr, whhr, bihr, bhhr, cw1r, cb1r, cw2r, cb2r,
              out_ref):
    wih = wihr[...]
    whh = whhr[...]
    bih = bihr[...]
    bhh = bhhr[...]
    h = jnp.zeros((BLK, H), jnp.float32)
    for xr in (x0r, x1r, x2r):
        xt = xr[...]
        gi = jnp.dot(xt, wih, preferred_element_type=jnp.float32) + bih
        gh = jnp.dot(h, whh, preferred_element_type=jnp.float32) + bhh
        rg = jax.nn.sigmoid(gi[:, :H] + gh[:, :H])
        zg = jax.nn.sigmoid(gi[:, H:2 * H] + gh[:, H:2 * H])
        ng = jnp.tanh(gi[:, 2 * H:] + rg * gh[:, 2 * H:])
        h = (1.0 - zg) * ng + zg * h
    zf = jnp.maximum(
        jnp.dot(h, cw1r[...], preferred_element_type=jnp.float32) + cb1r[...],
        0.0)
    out_ref[...] = (jnp.dot(zf, cw2r[...], preferred_element_type=jnp.float32)
                    + cb2r[...])


def _gru_cls_tc(h0, h1, h2, wihT, whhT, bih, bhh, cw1, cb1, cw2p, cb2):
    full = lambda shape: pl.BlockSpec(shape, lambda i: tuple(0 for _ in shape))
    blk = pl.BlockSpec((BLK, H), lambda i: (i, 0))
    return pl.pallas_call(
        _gru_body,
        grid=(NPAD // BLK,),
        in_specs=[blk, blk, blk,
                  full((H, 3 * H)), full((H, 3 * H)),
                  full((1, 3 * H)), full((1, 3 * H)),
                  full((H, H)), full((1, H)), full((H, H)), full((1, H))],
        out_specs=pl.BlockSpec((BLK, H), lambda i: (i, 0)),
        out_shape=jax.ShapeDtypeStruct((NPAD, H), jnp.float32),
    )(h0, h1, h2, wihT, whhT, bih, bhh, cw1, cb1, cw2p, cb2)


def kernel(feat_seq, edge_src, edge_dst, edge_w, W1, b1, W2, b2,
           gru_Wih, gru_Whh, gru_bih, gru_bhh, cls_W1, cls_b1, cls_W2,
           cls_b2):
    z2 = jnp.zeros((NPAD, HD), jnp.float32)
    z1 = jnp.zeros((NPAD,), jnp.float32)
    ones = jnp.ones((C,), jnp.float32)
    b1s = (b1[0] + b1[1]).reshape(1, H)
    b2s = (b2[0] + b2[1]).reshape(1, H)

    sc_n_deg = _make_sc_pass(N, True)
    sc_p = _make_sc_pass(NPAD, False)

    npad_e = EPAD - E
    pad_idx = jnp.arange(npad_e, dtype=jnp.int32)
    pad_src = jnp.broadcast_to((pad_idx % N)[None, None], (T, R, npad_e))
    pad_dst = jnp.broadcast_to((N + pad_idx % (NPAD - N))[None, None],
                               (T, R, npad_e))
    src_all = jnp.concatenate([edge_src, pad_src], axis=2)
    dst_all = jnp.concatenate([edge_dst, pad_dst], axis=2)
    w_all = jnp.concatenate(
        [edge_w, jnp.zeros((T, R, npad_e), jnp.float32)], axis=2)

    hs = []
    for t in range(T):
        src4 = src_all[t].reshape(R, NTILE, NCH, C)
        dst4 = dst_all[t].reshape(R, NTILE, NCH, C)
        w3 = w_all[t].reshape(R, NTILE, EPT)
        x = feat_seq[t]
        a1, deg = sc_n_deg(x[:, :HD], x[:, HD:], src4, dst4, w3, z2, z1,
                           ones)
        hl1 = _conv_tc(a1, deg, W1, b1s)
        a2 = sc_p(hl1[:, :HD], hl1[:, HD:], src4, dst4, w3, z2)
        hs.append(_conv_tc(a2, deg, W2, b2s))

    wihT = gru_Wih.T
    whhT = gru_Whh.T
    bih = gru_bih.reshape(1, 3 * H)
    bhh = gru_bhh.reshape(1, 3 * H)
    cb1 = cls_b1.reshape(1, H)
    cw2p = jnp.pad(cls_W2, ((0, 0), (0, H - 1)))
    cb2 = jnp.broadcast_to(cls_b2.reshape(1, 1), (1, H))
    logits = _gru_cls_tc(hs[0], hs[1], hs[2], wihT, whhT, bih, bhh,
                         cls_W1, cb1, cw2p, cb2)
    return logits[:N, 0]


# speedup vs baseline: 3.2392x; 1.0005x over previous
import functools

import jax
import jax.numpy as jnp
from jax import lax
from jax.experimental import pallas as pl
from jax.experimental.pallas import tpu as pltpu
from jax.experimental.pallas import tpu_sc as plsc

T, R, N, E, D, H = 3, 2, 10000, 320000, 128, 128
HD = D // 2
NTILE = 16
NPAD = 10240
RPT = NPAD // NTILE
C = 112
EPAD = 320768
EPT = EPAD // NTILE
NCH = EPT // C

_BCAST_DNUMS = lax.GatherDimensionNumbers(
    offset_dims=(), collapsed_slice_dims=(0,), start_index_map=(0,))


def _bcast_lane(vec, lane):
    idx = jnp.full((16, 1), lane, jnp.int32)
    return lax.gather(vec, idx, _BCAST_DNUMS, (1,),
                      mode=lax.GatherScatterMode.PROMISE_IN_BOUNDS)


def _scale_chunk(rows_ref, w1d, k):
    for g in range(C // 16):
        wvec = w1d[pl.ds(k * C + g * 16, 16)]
        for u in range(16):
            wb = _bcast_lane(wvec, u)
            e = g * 16 + u
            for j in range(HD // 16):
                sl = pl.ds(j * 16, 16)
                rows_ref[e, sl] = rows_ref[e, sl] * wb


@functools.lru_cache(maxsize=None)
def _make_sc_pass(nx, with_deg):
    mesh = plsc.VectorSubcoreMesh(core_axis_name="c", subcore_axis_name="s")

    agg_t = jax.ShapeDtypeStruct((R, NPAD, D), jnp.float32)
    out_type = agg_t
    scratch = [
        pltpu.VMEM_SHARED((NPAD, HD), jnp.float32),
        pltpu.VMEM((NCH, C), jnp.int32),
        pltpu.VMEM((NCH, C), jnp.int32),
        pltpu.VMEM((EPT,), jnp.float32),
        pltpu.VMEM((4, C, HD), jnp.float32),
        pltpu.SemaphoreType.DMA((4,)),
        pltpu.SemaphoreType.DMA((4,)),
    ]
    if with_deg:
        out_type = [agg_t, jax.ShapeDtypeStruct((R, NPAD), jnp.float32)]
        scratch += [
            pltpu.VMEM_SHARED((NPAD,), jnp.float32),
            pltpu.VMEM((C,), jnp.float32),
        ]

    def sc_pass(*refs):
        if with_deg:
            (xlo_hbm, xhi_hbm, src_hbm, dst_hbm, w_hbm, z2_hbm, z1_hbm,
             ones_hbm, agg_out, deg_out, acc, sidx, didx, w1d, rows3, gsem,
             ssem, dacc, ones_v) = refs
        else:
            (xlo_hbm, xhi_hbm, src_hbm, dst_hbm, w_hbm, z2_hbm,
             agg_out, acc, sidx, didx, w1d, rows3, gsem, ssem) = refs
        c = lax.axis_index("c")
        s = lax.axis_index("s")
        row0 = s * RPT

        pltpu.sync_copy(src_hbm.at[c, s], sidx)
        pltpu.sync_copy(dst_hbm.at[c, s], didx)
        pltpu.sync_copy(w_hbm.at[c, s], w1d)
        if with_deg:
            pltpu.sync_copy(ones_hbm, ones_v)
            pltpu.sync_copy(z1_hbm.at[pl.ds(row0, RPT)],
                            dacc.at[pl.ds(row0, RPT)])

        def run_half(x_hbm, half, do_deg):
            pltpu.sync_copy(z2_hbm.at[pl.ds(row0, RPT)],
                            acc.at[pl.ds(row0, RPT)])
            plsc.subcore_barrier()

            def issue_gather(k, b):
                pltpu.async_copy(x_hbm.at[sidx.at[k]], rows3.at[b],
                                 gsem.at[b])

            def wait_gather(k, b):
                pltpu.make_async_copy(x_hbm.at[sidx.at[k]], rows3.at[b],
                                      gsem.at[b]).wait()

            def issue_scatter(k, b):
                pltpu.async_copy(rows3.at[b], acc.at[didx.at[k]], ssem.at[b],
                                 add=True)
                if do_deg:
                    pltpu.async_copy(ones_v, dacc.at[didx.at[k]], ssem.at[b],
                                     add=True)

            def wait_scatter(k, b):
                pltpu.make_async_copy(rows3.at[b], acc.at[didx.at[k]],
                                      ssem.at[b]).wait()
                if do_deg:
                    pltpu.make_async_copy(ones_v, dacc.at[didx.at[k]],
                                          ssem.at[b]).wait()

            issue_gather(0, 0)
            issue_gather(1, 1)

            def chunk_body(k, carry):
                b = lax.rem(k, 4)
                b2 = lax.rem(k + 2, 4)

                @pl.when(k >= 2)
                def _():
                    wait_scatter(k, b2)

                @pl.when(k + 2 < NCH)
                def _():
                    issue_gather(k + 2, b2)

                wait_gather(k, b)
                _scale_chunk(rows3.at[b], w1d, k)
                issue_scatter(k, b)
                return carry

            lax.fori_loop(0, NCH, chunk_body, 0)

            def drain_body(i, carry):
                wait_scatter(NCH - 1, lax.rem(NCH - 2 + i, 4))
                return carry

            lax.fori_loop(0, 2, drain_body, 0)
            plsc.subcore_barrier()

            pltpu.sync_copy(
                acc.at[pl.ds(row0, RPT)],
                agg_out.at[c, pl.ds(row0, RPT), pl.ds(half * HD, HD)])
            if do_deg:
                pltpu.sync_copy(dacc.at[pl.ds(row0, RPT)],
                                deg_out.at[c, pl.ds(row0, RPT)])
            plsc.subcore_barrier()

        run_half(xlo_hbm, 0, with_deg)
        run_half(xhi_hbm, 1, False)

    return pl.kernel(sc_pass, out_type=out_type, mesh=mesh,
                     scratch_types=scratch,
                     compiler_params=pltpu.CompilerParams(
                         use_tc_tiling_on_sc=False))


BLK = 1024


def _conv_body(agg_ref, deg_ref, w_ref, b_ref, out_ref):
    dg = jnp.maximum(deg_ref[...], 1.0)
    a = agg_ref[...]
    out = b_ref[...]
    for r in range(R):
        inv = (1.0 / dg[r])[:, None]
        out = out + jnp.dot(a[r] * inv, w_ref[r],
                            preferred_element_type=jnp.float32)
    out_ref[...] = jnp.maximum(out, 0.0)


def _conv_tc(agg, deg, w, b):
    return pl.pallas_call(
        _conv_body,
        grid=(NPAD // BLK,),
        in_specs=[
            pl.BlockSpec((R, BLK, D), lambda i: (0, i, 0)),
            pl.BlockSpec((R, BLK), lambda i: (0, i)),
            pl.BlockSpec((R, D, H), lambda i: (0, 0, 0)),
            pl.BlockSpec((1, H), lambda i: (0, 0)),
        ],
        out_specs=pl.BlockSpec((BLK, H), lambda i: (i, 0)),
        out_shape=jax.ShapeDtypeStruct((NPAD, H), jnp.float32),
    )(agg, deg, w, b)


def _gru_body(x0r, x1r, x2r, wihr, whhr, bihr, bhhr, cw1r, cb1r, cw2r, cb2r,
              out_ref):
    wih = wihr[...]
    whh = whhr[...]
    bih = bihr[...]
    bhh = bhhr[...]
    h = jnp.zeros((BLK, H), jnp.float32)
    for xr in (x0r, x1r, x2r):
        xt = xr[...]
        gi = jnp.dot(xt, wih, preferred_element_type=jnp.float32) + bih
        gh = jnp.dot(h, whh, preferred_element_type=jnp.float32) + bhh
        rg = jax.nn.sigmoid(gi[:, :H] + gh[:, :H])
        zg = jax.nn.sigmoid(gi[:, H:2 * H] + gh[:, H:2 * H])
        ng = jnp.tanh(gi[:, 2 * H:] + rg * gh[:, 2 * H:])
        h = (1.0 - zg) * ng + zg * h
    zf = jnp.maximum(
        jnp.dot(h, cw1r[...], preferred_element_type=jnp.float32) + cb1r[...],
        0.0)
    out_ref[...] = (jnp.dot(zf, cw2r[...], preferred_element_type=jnp.float32)
                    + cb2r[...])


def _gru_cls_tc(h0, h1, h2, wihT, whhT, bih, bhh, cw1, cb1, cw2p, cb2):
    full = lambda shape: pl.BlockSpec(shape, lambda i: tuple(0 for _ in shape))
    blk = pl.BlockSpec((BLK, H), lambda i: (i, 0))
    return pl.pallas_call(
        _gru_body,
        grid=(NPAD // BLK,),
        in_specs=[blk, blk, blk,
                  full((H, 3 * H)), full((H, 3 * H)),
                  full((1, 3 * H)), full((1, 3 * H)),
                  full((H, H)), full((1, H)), full((H, H)), full((1, H))],
        out_specs=pl.BlockSpec((BLK, H), lambda i: (i, 0)),
        out_shape=jax.ShapeDtypeStruct((NPAD, H), jnp.float32),
    )(h0, h1, h2, wihT, whhT, bih, bhh, cw1, cb1, cw2p, cb2)


def kernel(feat_seq, edge_src, edge_dst, edge_w, W1, b1, W2, b2,
           gru_Wih, gru_Whh, gru_bih, gru_bhh, cls_W1, cls_b1, cls_W2,
           cls_b2):
    z2 = jnp.zeros((NPAD, HD), jnp.float32)
    z1 = jnp.zeros((NPAD,), jnp.float32)
    ones = jnp.ones((C,), jnp.float32)
    b1s = (b1[0] + b1[1]).reshape(1, H)
    b2s = (b2[0] + b2[1]).reshape(1, H)

    sc_n_deg = _make_sc_pass(N, True)
    sc_p = _make_sc_pass(NPAD, False)

    npad_e = EPAD - E
    pad_idx = jnp.arange(npad_e, dtype=jnp.int32)
    pad_src = jnp.broadcast_to((pad_idx % N)[None, None], (T, R, npad_e))
    pad_dst = jnp.broadcast_to((N + pad_idx % (NPAD - N))[None, None],
                               (T, R, npad_e))
    src_all = jnp.concatenate([edge_src, pad_src], axis=2)
    dst_all = jnp.concatenate([edge_dst, pad_dst], axis=2)
    w_all = jnp.concatenate(
        [edge_w, jnp.zeros((T, R, npad_e), jnp.float32)], axis=2)

    hs = []
    for t in range(T):
        src4 = src_all[t].reshape(R, NTILE, NCH, C)
        dst4 = dst_all[t].reshape(R, NTILE, NCH, C)
        w3 = w_all[t].reshape(R, NTILE, EPT)
        x = feat_seq[t]
        a1, deg = sc_n_deg(x[:, :HD], x[:, HD:], src4, dst4, w3, z2, z1,
                           ones)
        hl1 = _conv_tc(a1, deg, W1, b1s)
        a2 = sc_p(hl1[:, :HD], hl1[:, HD:], src4, dst4, w3, z2)
        hs.append(_conv_tc(a2, deg, W2, b2s))

    wihT = gru_Wih.T
    whhT = gru_Whh.T
    bih = gru_bih.reshape(1, 3 * H)
    bhh = gru_bhh.reshape(1, 3 * H)
    cb1 = cls_b1.reshape(1, H)
    cw2p = jnp.pad(cls_W2, ((0, 0), (0, H - 1)))
    cb2 = jnp.broadcast_to(cls_b2.reshape(1, 1), (1, H))
    logits = _gru_cls_tc(hs[0], hs[1], hs[2], wihT, whhT, bih, bhh,
                         cls_W1, cb1, cw2p, cb2)
    return logits[:N, 0]
